# baseline (speedup 1.0000x reference)
import functools

import jax
import jax.numpy as jnp
from jax import lax
from jax.experimental import pallas as pl
from jax.experimental.pallas import tpu as pltpu
from jax.experimental.pallas import tpu_sc as plsc

D_IN = 4096
D_OUT = 4096
NUMEL = D_OUT * D_IN

NC = 2
NS = 16
NW = NC * NS

WSH = 15
WIN = 1 << WSH
BINS = NUMEL // WIN
PW = 49152
PADTOT = PW * NW
STAGE = 4096
NSTAGE = PW // STAGE
CAP_SEG = 192
OFFS_ROW = 528
WPW = BINS // NW

_MESH = plsc.VectorSubcoreMesh(
    core_axis_name="c", subcore_axis_name="s", num_cores=NC, num_subcores=NS
)
_PARAMS = pltpu.CompilerParams(needs_layout_passes=False)


def _bucket_body(idx_hbm, val_hbm, pi_hbm, pv_hbm, offs_hbm,
                 stage_i, stage_v, outi_v, outv_v, hist_v, offs_st, sem):
    c = lax.axis_index("c")
    s = lax.axis_index("s")
    wid = s * NC + c
    base = wid * PW

    def zero_step(i, _):
        hist_v[pl.ds(i * 16, 16)] = jnp.zeros((16,), jnp.int32)
        return ()

    lax.fori_loop(0, BINS // 16, zero_step, ())

    def hist_chunk(sc_i, _):
        pltpu.sync_copy(idx_hbm.at[pl.ds(base + sc_i * STAGE, STAGE)], stage_i)

        def hist_vreg(k, _):
            v = stage_i[pl.ds(k * 16, 16)]
            b = lax.shift_right_logical(v, WSH)
            r, lastm = plsc.scan_count(b)
            plsc.addupdate_scatter(hist_v, [b], r, mask=lastm)
            return ()

        lax.fori_loop(0, STAGE // 16, hist_vreg, ())
        return ()

    lax.fori_loop(0, NSTAGE, hist_chunk, ())

    def scan_step(i, carry):
        v = hist_v[pl.ds(i * 16, 16)]
        csum = plsc.cumsum(v)
        ex = csum - v + carry
        hist_v[pl.ds(i * 16, 16)] = ex
        offs_st[pl.ds(i * 16, 16)] = ex
        return carry + csum[15]

    lax.fori_loop(0, BINS // 16, scan_step, jnp.int32(0))
    offs_st[pl.ds(BINS, 16)] = jnp.full((16,), PW, jnp.int32)
    pltpu.sync_copy(offs_st, offs_hbm.at[wid])

    def perm_chunk(sc_i, _):
        pltpu.sync_copy(idx_hbm.at[pl.ds(base + sc_i * STAGE, STAGE)], stage_i)
        pltpu.sync_copy(val_hbm.at[pl.ds(base + sc_i * STAGE, STAGE)], stage_v)

        def perm_vreg(k, _):
            v = stage_i[pl.ds(k * 16, 16)]
            w = stage_v[pl.ds(k * 16, 16)]
            b = lax.shift_right_logical(v, WSH)
            r, lastm = plsc.scan_count(b)
            cur = plsc.load_gather(hist_v, [b])
            slot = cur + r - 1
            plsc.store_scatter(outi_v, [slot], v)
            plsc.store_scatter(outv_v, [slot], w)
            plsc.addupdate_scatter(hist_v, [b], r, mask=lastm)
            return ()

        lax.fori_loop(0, STAGE // 16, perm_vreg, ())
        return ()

    lax.fori_loop(0, NSTAGE, perm_chunk, ())
    pltpu.sync_copy(outi_v, pi_hbm.at[pl.ds(base, PW)])
    pltpu.sync_copy(outv_v, pv_hbm.at[pl.ds(base, PW)])


_bucket = pl.kernel(
    _bucket_body,
    out_type=[
        jax.ShapeDtypeStruct((PADTOT + CAP_SEG,), jnp.int32),
        jax.ShapeDtypeStruct((PADTOT + CAP_SEG,), jnp.float32),
        jax.ShapeDtypeStruct((NW, OFFS_ROW), jnp.int32),
    ],
    mesh=_MESH,
    compiler_params=_PARAMS,
    scratch_types=[
        pltpu.VMEM((STAGE,), jnp.int32),
        pltpu.VMEM((STAGE,), jnp.float32),
        pltpu.VMEM((PW,), jnp.int32),
        pltpu.VMEM((PW,), jnp.float32),
        pltpu.VMEM((BINS,), jnp.int32),
        pltpu.VMEM((OFFS_ROW,), jnp.int32),
        pltpu.SemaphoreType.DMA,
    ],
)


def _merge_body(base_band, w_hbm, pi_hbm, pv_hbm, offs_hbm, out_hbm,
                offs_v, wbuf0, wbuf1, *scratch):
    segi = [scratch[:NW], scratch[NW : 2 * NW]]
    segv = [scratch[2 * NW : 3 * NW], scratch[3 * NW : 4 * NW]]
    wbuf = [wbuf0, wbuf1]
    semw = [scratch[4 * NW], scratch[4 * NW + 1]]
    sems = [scratch[4 * NW + 2], scratch[4 * NW + 3]]
    semo = [scratch[4 * NW + 4], scratch[4 * NW + 5]]
    c = lax.axis_index("c")
    s = lax.axis_index("s")
    wid = s * NC + c
    pltpu.sync_copy(offs_hbm, offs_v)
    iota = lax.iota(jnp.int32, 16)

    def stage(v, p):
        wcp = pltpu.async_copy(w_hbm.at[pl.ds(v * 8, 8)], wbuf[p], semw[p])
        vvec = jnp.full((16,), v, jnp.int32)
        s_lo = plsc.load_gather(offs_v, [iota, vvec])
        s_hi = plsc.load_gather(offs_v, [iota + 16, vvec])
        e_lo = plsc.load_gather(offs_v, [iota, vvec + 1])
        e_hi = plsc.load_gather(offs_v, [iota + 16, vvec + 1])
        scps = []
        counts = []
        for t in range(NW):
            st = (s_lo if t < 16 else s_hi)[t % 16]
            en = (e_lo if t < 16 else e_hi)[t % 16]
            st8 = pl.multiple_of(jnp.bitwise_and(st, jnp.int32(-8)), 8)
            counts.append((st - st8, en - st))
            scps.append(pltpu.async_copy(
                pi_hbm.at[pl.ds(t * PW + st8, CAP_SEG)], segi[p][t], sems[p]))
            scps.append(pltpu.async_copy(
                pv_hbm.at[pl.ds(t * PW + st8, CAP_SEG)], segv[p][t], sems[p]))
        return wcp, scps, counts

    def apply(v, p, counts):
        for t in range(NW):
            o0, n_t = counts[t]
            hi = o0 + n_t

            def apply_vreg(j, _, t=t, o0=o0, hi=hi, p=p):
                pos = iota + j * 16
                iv = segi[p][t][pl.ds(j * 16, 16)]
                vv = segv[p][t][pl.ds(j * 16, 16)]
                r8 = jnp.bitwise_and(lax.shift_right_logical(iv, 12), 7)
                cc = jnp.bitwise_and(iv, D_IN - 1)
                mask = jnp.logical_and(pos >= o0, pos < hi)
                plsc.store_scatter(wbuf[p], [r8, cc], vv, mask=mask)
                return ()

            lax.fori_loop(0, (hi + 15) >> 4, apply_vreg, ())
        return pltpu.async_copy(
            wbuf[p], out_hbm.at[pl.ds((v - base_band) * 8, 8)], semo[p])

    def pair_step(i, _):
        v0 = base_band + wid * (WPW // 2) + i * 2
        wcp0, scps0, counts0 = stage(v0, 0)
        wcp1, scps1, counts1 = stage(v0 + 1, 1)
        wcp0.wait()
        for cp in scps0:
            cp.wait()
        ocp0 = apply(v0, 0, counts0)
        wcp1.wait()
        for cp in scps1:
            cp.wait()
        ocp1 = apply(v0 + 1, 1, counts1)
        ocp0.wait()
        ocp1.wait()
        return ()

    lax.fori_loop(0, WPW // 4, pair_step, ())


def _make_merge(base_band):
    return pl.kernel(
        functools.partial(_merge_body, base_band),
        out_type=jax.ShapeDtypeStruct((D_OUT // 2, D_IN), jnp.float32),
        mesh=_MESH,
        compiler_params=_PARAMS,
        scratch_types=(
            [pltpu.VMEM((NW, OFFS_ROW), jnp.int32),
             pltpu.VMEM((8, D_IN), jnp.float32),
             pltpu.VMEM((8, D_IN), jnp.float32)]
            + [pltpu.VMEM((CAP_SEG,), jnp.int32) for _ in range(2 * NW)]
            + [pltpu.VMEM((CAP_SEG,), jnp.float32) for _ in range(2 * NW)]
            + [pltpu.SemaphoreType.DMA for _ in range(6)]
        ),
    )


_merge_lo = _make_merge(0)
_merge_hi = _make_merge(BINS // 2)


def _mm_body(x_ref, w_ref, b_ref, o_ref):
    xb = x_ref[...].astype(jnp.bfloat16)
    wb = w_ref[...].astype(jnp.bfloat16)
    acc = lax.dot_general(
        xb, wb, (((1,), (1,)), ((), ())), preferred_element_type=jnp.float32
    )
    o_ref[...] = acc + b_ref[...]


def _tc_matmul(x, w, b2d):
    n_blk = w.shape[0] // 256
    return pl.pallas_call(
        _mm_body,
        grid=(n_blk,),
        in_specs=[
            pl.BlockSpec((256, D_IN), lambda i: (0, 0)),
            pl.BlockSpec((256, D_IN), lambda i: (i, 0)),
            pl.BlockSpec((1, 256), lambda i: (0, i)),
        ],
        out_specs=pl.BlockSpec((256, 256), lambda i: (0, i)),
        out_shape=jax.ShapeDtypeStruct((256, w.shape[0]), jnp.float32),
    )(x, w, b2d)


def kernel(x, W_flat, b, flip_vals, flip_idx):
    n = flip_idx.shape[0]
    pad = PADTOT - n
    idx = flip_idx.astype(jnp.int32)
    idx_p = jnp.concatenate([idx, idx[:pad]])
    val_p = jnp.concatenate([flip_vals, flip_vals[:pad]])

    pi, pv, offs = _bucket(idx_p, val_p)
    w2d = W_flat.reshape(D_OUT, D_IN)
    half = D_OUT // 2
    w_lo = _merge_lo(w2d, pi, pv, offs)
    w_hi = _merge_hi(w2d, pi, pv, offs)
    out_lo = _tc_matmul(x, w_lo, b[:half].reshape(1, half))
    out_hi = _tc_matmul(x, w_hi, b[half:].reshape(1, half))
    return jnp.concatenate([out_lo, out_hi], axis=1)


# speedup vs baseline: 1.0395x; 1.0395x over previous
import functools

import jax
import jax.numpy as jnp
from jax import lax
from jax.experimental import pallas as pl
from jax.experimental.pallas import tpu as pltpu
from jax.experimental.pallas import tpu_sc as plsc

D_IN = 4096
D_OUT = 4096
NUMEL = D_OUT * D_IN

NC = 2
NS = 16
NW = NC * NS

WSH = 15
WIN = 1 << WSH
BINS = NUMEL // WIN
PW = 49152
PADTOT = PW * NW
STAGE = 4096
NSTAGE = PW // STAGE
CAP_SEG = 192
OFFS_ROW = 528
WPW = BINS // NW

_MESH = plsc.VectorSubcoreMesh(
    core_axis_name="c", subcore_axis_name="s", num_cores=NC, num_subcores=NS
)
_PARAMS = pltpu.CompilerParams(needs_layout_passes=False)


def _bucket_body(idx_hbm, val_hbm, pi_hbm, pv_hbm, offs_hbm,
                 stage_i, stage_v, outi_v, outv_v, hist_v, offs_st, sem):
    c = lax.axis_index("c")
    s = lax.axis_index("s")
    wid = s * NC + c
    base = wid * PW

    def zero_step(i, _):
        hist_v[pl.ds(i * 16, 16)] = jnp.zeros((16,), jnp.int32)
        return ()

    lax.fori_loop(0, BINS // 16, zero_step, ())

    def hist_chunk(sc_i, _):
        pltpu.sync_copy(idx_hbm.at[pl.ds(base + sc_i * STAGE, STAGE)], stage_i)

        def hist_vreg(k, _):
            v = stage_i[pl.ds(k * 16, 16)]
            b = lax.shift_right_logical(v, WSH)
            r, lastm = plsc.scan_count(b)
            plsc.addupdate_scatter(hist_v, [b], r, mask=lastm)
            return ()

        lax.fori_loop(0, STAGE // 16, hist_vreg, ())
        return ()

    lax.fori_loop(0, NSTAGE, hist_chunk, ())

    def scan_step(i, carry):
        v = hist_v[pl.ds(i * 16, 16)]
        csum = plsc.cumsum(v)
        ex = csum - v + carry
        hist_v[pl.ds(i * 16, 16)] = ex
        offs_st[pl.ds(i * 16, 16)] = ex
        return carry + csum[15]

    lax.fori_loop(0, BINS // 16, scan_step, jnp.int32(0))
    offs_st[pl.ds(BINS, 16)] = jnp.full((16,), PW, jnp.int32)
    pltpu.sync_copy(offs_st, offs_hbm.at[wid])

    def perm_chunk(sc_i, _):
        pltpu.sync_copy(idx_hbm.at[pl.ds(base + sc_i * STAGE, STAGE)], stage_i)
        pltpu.sync_copy(val_hbm.at[pl.ds(base + sc_i * STAGE, STAGE)], stage_v)

        def perm_vreg(k, _):
            v = stage_i[pl.ds(k * 16, 16)]
            w = stage_v[pl.ds(k * 16, 16)]
            b = lax.shift_right_logical(v, WSH)
            r, lastm = plsc.scan_count(b)
            cur = plsc.load_gather(hist_v, [b])
            slot = cur + r - 1
            plsc.store_scatter(outi_v, [slot], v)
            plsc.store_scatter(outv_v, [slot], w)
            plsc.addupdate_scatter(hist_v, [b], r, mask=lastm)
            return ()

        lax.fori_loop(0, STAGE // 16, perm_vreg, ())
        return ()

    lax.fori_loop(0, NSTAGE, perm_chunk, ())
    pltpu.sync_copy(outi_v, pi_hbm.at[pl.ds(base, PW)])
    pltpu.sync_copy(outv_v, pv_hbm.at[pl.ds(base, PW)])


_bucket = pl.kernel(
    _bucket_body,
    out_type=[
        jax.ShapeDtypeStruct((PADTOT + CAP_SEG,), jnp.int32),
        jax.ShapeDtypeStruct((PADTOT + CAP_SEG,), jnp.float32),
        jax.ShapeDtypeStruct((NW, OFFS_ROW), jnp.int32),
    ],
    mesh=_MESH,
    compiler_params=_PARAMS,
    scratch_types=[
        pltpu.VMEM((STAGE,), jnp.int32),
        pltpu.VMEM((STAGE,), jnp.float32),
        pltpu.VMEM((PW,), jnp.int32),
        pltpu.VMEM((PW,), jnp.float32),
        pltpu.VMEM((BINS,), jnp.int32),
        pltpu.VMEM((OFFS_ROW,), jnp.int32),
        pltpu.SemaphoreType.DMA,
    ],
)


def _merge_body(w_hbm, pi_hbm, pv_hbm, offs_hbm, out_hbm,
                offs_v, wbuf0, wbuf1, *scratch):
    segi = [scratch[:NW], scratch[NW : 2 * NW]]
    segv = [scratch[2 * NW : 3 * NW], scratch[3 * NW : 4 * NW]]
    wbuf = [wbuf0, wbuf1]
    semw = [scratch[4 * NW], scratch[4 * NW + 1]]
    sems = [scratch[4 * NW + 2], scratch[4 * NW + 3]]
    semo = [scratch[4 * NW + 4], scratch[4 * NW + 5]]
    c = lax.axis_index("c")
    s = lax.axis_index("s")
    wid = s * NC + c
    pltpu.sync_copy(offs_hbm, offs_v)
    iota = lax.iota(jnp.int32, 16)

    def stage(v, p):
        wcp = pltpu.async_copy(w_hbm.at[pl.ds(v * 8, 8)], wbuf[p], semw[p])
        vvec = jnp.full((16,), v, jnp.int32)
        s_lo = plsc.load_gather(offs_v, [iota, vvec])
        s_hi = plsc.load_gather(offs_v, [iota + 16, vvec])
        e_lo = plsc.load_gather(offs_v, [iota, vvec + 1])
        e_hi = plsc.load_gather(offs_v, [iota + 16, vvec + 1])
        scps = []
        counts = []
        for t in range(NW):
            st = (s_lo if t < 16 else s_hi)[t % 16]
            en = (e_lo if t < 16 else e_hi)[t % 16]
            st8 = pl.multiple_of(jnp.bitwise_and(st, jnp.int32(-8)), 8)
            counts.append((st - st8, en - st))
            scps.append(pltpu.async_copy(
                pi_hbm.at[pl.ds(t * PW + st8, CAP_SEG)], segi[p][t], sems[p]))
            scps.append(pltpu.async_copy(
                pv_hbm.at[pl.ds(t * PW + st8, CAP_SEG)], segv[p][t], sems[p]))
        return wcp, scps, counts

    def apply(v, p, counts):
        for t in range(NW):
            o0, n_t = counts[t]
            hi = o0 + n_t

            def apply_vreg(j, _, t=t, o0=o0, hi=hi, p=p):
                pos = iota + j * 16
                iv = segi[p][t][pl.ds(j * 16, 16)]
                vv = segv[p][t][pl.ds(j * 16, 16)]
                r8 = jnp.bitwise_and(lax.shift_right_logical(iv, 12), 7)
                cc = jnp.bitwise_and(iv, D_IN - 1)
                mask = jnp.logical_and(pos >= o0, pos < hi)
                plsc.store_scatter(wbuf[p], [r8, cc], vv, mask=mask)
                return ()

            lax.fori_loop(0, (hi + 15) >> 4, apply_vreg, ())
        return pltpu.async_copy(wbuf[p], out_hbm.at[pl.ds(v * 8, 8)], semo[p])

    def pair_step(i, _):
        v0 = wid * WPW + i * 2
        wcp0, scps0, counts0 = stage(v0, 0)
        wcp1, scps1, counts1 = stage(v0 + 1, 1)
        wcp0.wait()
        for cp in scps0:
            cp.wait()
        ocp0 = apply(v0, 0, counts0)
        wcp1.wait()
        for cp in scps1:
            cp.wait()
        ocp1 = apply(v0 + 1, 1, counts1)
        ocp0.wait()
        ocp1.wait()
        return ()

    lax.fori_loop(0, WPW // 2, pair_step, ())


_merge = pl.kernel(
    _merge_body,
    out_type=jax.ShapeDtypeStruct((D_OUT, D_IN), jnp.float32),
    mesh=_MESH,
    compiler_params=_PARAMS,
    scratch_types=(
        [pltpu.VMEM((NW, OFFS_ROW), jnp.int32),
         pltpu.VMEM((8, D_IN), jnp.float32),
         pltpu.VMEM((8, D_IN), jnp.float32)]
        + [pltpu.VMEM((CAP_SEG,), jnp.int32) for _ in range(2 * NW)]
        + [pltpu.VMEM((CAP_SEG,), jnp.float32) for _ in range(2 * NW)]
        + [pltpu.SemaphoreType.DMA for _ in range(6)]
    ),
)


def _mm_body(x_ref, w_ref, b_ref, o_ref):
    xb = x_ref[...].astype(jnp.bfloat16)
    wb = w_ref[...].astype(jnp.bfloat16)
    acc = lax.dot_general(
        xb, wb, (((1,), (1,)), ((), ())), preferred_element_type=jnp.float32
    )
    o_ref[...] = acc + b_ref[...]


def _tc_matmul(x, w, b2d):
    n_blk = w.shape[0] // 256
    return pl.pallas_call(
        _mm_body,
        grid=(n_blk,),
        in_specs=[
            pl.BlockSpec((256, D_IN), lambda i: (0, 0)),
            pl.BlockSpec((256, D_IN), lambda i: (i, 0)),
            pl.BlockSpec((1, 256), lambda i: (0, i)),
        ],
        out_specs=pl.BlockSpec((256, 256), lambda i: (0, i)),
        out_shape=jax.ShapeDtypeStruct((256, w.shape[0]), jnp.float32),
    )(x, w, b2d)


def kernel(x, W_flat, b, flip_vals, flip_idx):
    n = flip_idx.shape[0]
    pad = PADTOT - n
    idx = flip_idx.astype(jnp.int32)
    idx_p = jnp.concatenate([idx, idx[:pad]])
    val_p = jnp.concatenate([flip_vals, flip_vals[:pad]])

    pi, pv, offs = _bucket(idx_p, val_p)
    w_mod = _merge(W_flat.reshape(D_OUT, D_IN), pi, pv, offs)
    return _tc_matmul(x, w_mod, b.reshape(1, D_OUT))
